# Initial kernel scaffold; baseline (speedup 1.0000x reference)
import jax
import jax.numpy as jnp
from jax import lax
from jax.experimental import pallas as pl
from jax.experimental.pallas import tpu as pltpu
from jax.experimental.pallas import tpu_sc as plsc

_NV = 10000
_NC = 42000
_K = 3
_B = 128

_NSUB = 16
_CPT = 2640
_NCP = _NSUB * _CPT
_STEPS = _CPT // 16
_NVP = 10048
_PAD = _NCP - _NC


def _sat_body(x_hbm, idx_hbm, w_hbm, out_hbm,
              x_v, idx_v, w_v, acc_v, shared, red_v, out_v):
    c = lax.axis_index("c")
    s = lax.axis_index("s")
    base = s * _CPT

    pltpu.sync_copy(x_hbm, x_v)
    for j in range(_K):
        pltpu.sync_copy(idx_hbm.at[j, pl.ds(base, _CPT)], idx_v.at[j])
        pltpu.sync_copy(w_hbm.at[j, pl.ds(base, _CPT)], w_v.at[j])

    def body(t, acc):
        off = t * 16
        pre = jnp.full((16,), float(_K - 1), dtype=jnp.float32)
        for j in range(_K):
            ij = idx_v[j, pl.ds(off, 16)]
            vj = plsc.load_gather(x_v, [ij])
            wj = w_v[j, pl.ds(off, 16)]
            pre = pre + wj * jnp.sign(vj)
        return acc + jnp.sign(pre)

    acc = lax.fori_loop(0, _STEPS, body, jnp.zeros((16,), jnp.float32))

    acc_v[...] = acc
    pltpu.sync_copy(acc_v, shared.at[s])
    plsc.subcore_barrier()

    @pl.when(jnp.logical_and(c == 0, s == 0))
    def _():
        pltpu.sync_copy(shared, red_v)
        tot = jnp.zeros((16,), jnp.float32)
        for si in range(_NSUB):
            tot = tot + red_v[si]
        total = jnp.sum(tot)
        thresh = float(_PAD + _NC - 1)
        outvec = jnp.sign(jnp.broadcast_to(total - thresh, (16,)))
        for k in range(_B // 16):
            out_v[pl.ds(k * 16, 16)] = outvec
        pltpu.sync_copy(out_v, out_hbm)


def kernel(input, emb_weight, or_weight, clause_idx):
    del input
    x_flat = jnp.pad(emb_weight.reshape(-1), (0, _NVP - _NV))
    idx_t = jnp.pad(clause_idx.T, ((0, 0), (0, _NCP - _NC)))
    w_t = jnp.pad(or_weight.T, ((0, 0), (0, _NCP - _NC)))

    mesh = plsc.VectorSubcoreMesh(core_axis_name="c", subcore_axis_name="s")
    f = pl.kernel(
        _sat_body,
        mesh=mesh,
        out_type=jax.ShapeDtypeStruct((_B,), jnp.float32),
        scratch_types=[
            pltpu.VMEM((_NVP,), jnp.float32),
            pltpu.VMEM((_K, _CPT), jnp.int32),
            pltpu.VMEM((_K, _CPT), jnp.float32),
            pltpu.VMEM((16,), jnp.float32),
            pltpu.VMEM_SHARED((_NSUB, 16), jnp.float32),
            pltpu.VMEM((_NSUB, 16), jnp.float32),
            pltpu.VMEM((_B,), jnp.float32),
        ],
    )
    return f(x_flat, idx_t, w_t)


# speedup vs baseline: 21.6541x; 21.6541x over previous
import jax
import jax.numpy as jnp
from jax import lax
from jax.experimental import pallas as pl
from jax.experimental.pallas import tpu as pltpu
from jax.experimental.pallas import tpu_sc as plsc

_NV = 10000
_NC = 42000
_K = 3
_B = 128

_NSUB = 16
_CPT = 2640
_NCP = _NSUB * _CPT
_STEPS = _CPT // 16
_NVP = 10048
_PAD = _NCP - _NC


def _sat_body(x_hbm, idx_hbm, w_hbm, out_hbm,
              x_v, idx_v, w_v, acc_v, shared, red_v, out_v):
    c = lax.axis_index("c")
    s = lax.axis_index("s")
    base = s * _CPT

    pltpu.sync_copy(x_hbm, x_v)
    for j in range(_K):
        pltpu.sync_copy(idx_hbm.at[pl.ds(j * _NCP + base, _CPT)],
                        idx_v.at[pl.ds(j * _CPT, _CPT)])
        pltpu.sync_copy(w_hbm.at[pl.ds(j * _NCP + base, _CPT)],
                        w_v.at[pl.ds(j * _CPT, _CPT)])

    def body(t, acc):
        off = t * 16
        pre = jnp.full((16,), float(_K - 1), dtype=jnp.float32)
        for j in range(_K):
            ij = idx_v[pl.ds(j * _CPT + off, 16)]
            vj = plsc.load_gather(x_v, [ij])
            wj = w_v[pl.ds(j * _CPT + off, 16)]
            pre = pre + wj * jnp.sign(vj)
        return acc + jnp.sign(pre)

    acc = lax.fori_loop(0, _STEPS, body, jnp.zeros((16,), jnp.float32))

    acc_v[...] = acc
    pltpu.sync_copy(acc_v, shared.at[s])
    plsc.subcore_barrier()

    @pl.when(jnp.logical_and(c == 0, s == 0))
    def _():
        pltpu.sync_copy(shared, red_v)
        tot = jnp.zeros((16,), jnp.float32)
        for si in range(_NSUB):
            tot = tot + red_v[si]
        total = jnp.sum(tot)
        thresh = float(_PAD + _NC - 1)
        outvec = jnp.sign(jnp.broadcast_to(total - thresh, (16,)))
        for k in range(_B // 16):
            out_v[pl.ds(k * 16, 16)] = outvec
        pltpu.sync_copy(out_v, out_hbm)


def kernel(input, emb_weight, or_weight, clause_idx):
    del input
    x_flat = jnp.pad(emb_weight.reshape(-1), (0, _NVP - _NV))
    idx_t = jnp.pad(clause_idx.T, ((0, 0), (0, _NCP - _NC))).reshape(-1)
    w_t = jnp.pad(or_weight.T, ((0, 0), (0, _NCP - _NC))).reshape(-1)

    mesh = plsc.VectorSubcoreMesh(core_axis_name="c", subcore_axis_name="s")
    f = pl.kernel(
        _sat_body,
        mesh=mesh,
        out_type=jax.ShapeDtypeStruct((_B,), jnp.float32),
        compiler_params=pltpu.CompilerParams(needs_layout_passes=False),
        scratch_types=[
            pltpu.VMEM((_NVP,), jnp.float32),
            pltpu.VMEM((_K * _CPT,), jnp.int32),
            pltpu.VMEM((_K * _CPT,), jnp.float32),
            pltpu.VMEM((16,), jnp.float32),
            pltpu.VMEM_SHARED((_NSUB, 16), jnp.float32),
            pltpu.VMEM((_NSUB, 16), jnp.float32),
            pltpu.VMEM((_B,), jnp.float32),
        ],
    )
    return f(x_flat, idx_t, w_t)


# speedup vs baseline: 23.7920x; 1.0987x over previous
import jax
import jax.numpy as jnp
from jax import lax
from jax.experimental import pallas as pl
from jax.experimental.pallas import tpu as pltpu
from jax.experimental.pallas import tpu_sc as plsc

_NV = 10000
_NC = 42000
_K = 3
_B = 128

_NSUB = 16
_CPT = 2640
_NCP = _NSUB * _CPT
_STEPS = _CPT // 16
_NVP = 10048
_PAD = _NCP - _NC


def _sat_body(x_hbm, idx_hbm, w_hbm, out_hbm,
              x_v, idx_v, w_v, acc_v, shared, red_v, out_v):
    c = lax.axis_index("c")
    s = lax.axis_index("s")
    base = s * _CPT

    pltpu.sync_copy(x_hbm, x_v)
    for j in range(_K):
        pltpu.sync_copy(idx_hbm.at[pl.ds(j * _NCP + base, _CPT)],
                        idx_v.at[pl.ds(j * _CPT, _CPT)])
        pltpu.sync_copy(w_hbm.at[pl.ds(j * _NCP + base, _CPT)],
                        w_v.at[pl.ds(j * _CPT, _CPT)])

    def body(t, acc):
        off = t * 16
        pre = jnp.full((16,), float(_K - 1), dtype=jnp.float32)
        for j in range(_K):
            ij = idx_v[pl.ds(j * _CPT + off, 16)]
            vj = plsc.load_gather(x_v, [ij])
            wj = w_v[pl.ds(j * _CPT + off, 16)]
            pre = pre + wj * jnp.sign(vj)
        return acc + jnp.sign(pre)

    acc = lax.fori_loop(0, _STEPS, body, jnp.zeros((16,), jnp.float32))

    acc_v[...] = acc
    pltpu.sync_copy(acc_v, shared.at[s])
    plsc.subcore_barrier()

    @pl.when(jnp.logical_and(c == 0, s == 0))
    def _():
        pltpu.sync_copy(shared, red_v)
        tot = jnp.zeros((16,), jnp.float32)
        for si in range(_NSUB):
            tot = tot + red_v[si]
        total = jnp.sum(tot)
        thresh = float(_PAD + _NC - 1)
        outvec = jnp.sign(jnp.broadcast_to(total - thresh, (16,)))
        for k in range(_B // 16):
            out_v[pl.ds(k * 16, 16)] = outvec
        pltpu.sync_copy(out_v, out_hbm)


def kernel(input, emb_weight, or_weight, clause_idx):
    del input
    x_flat = jnp.pad(emb_weight.reshape(-1), (0, _NVP - _NV))
    idx_t = jnp.pad(clause_idx.T, ((0, 0), (0, _NCP - _NC))).reshape(-1)
    w_t = jnp.pad(or_weight.T, ((0, 0), (0, _NCP - _NC))).reshape(-1)

    mesh = plsc.VectorSubcoreMesh(
        core_axis_name="c", subcore_axis_name="s", num_cores=1)
    f = pl.kernel(
        _sat_body,
        mesh=mesh,
        out_type=jax.ShapeDtypeStruct((_B,), jnp.float32),
        compiler_params=pltpu.CompilerParams(needs_layout_passes=False),
        scratch_types=[
            pltpu.VMEM((_NVP,), jnp.float32),
            pltpu.VMEM((_K * _CPT,), jnp.int32),
            pltpu.VMEM((_K * _CPT,), jnp.float32),
            pltpu.VMEM((16,), jnp.float32),
            pltpu.VMEM_SHARED((_NSUB, 16), jnp.float32),
            pltpu.VMEM((_NSUB, 16), jnp.float32),
            pltpu.VMEM((_B,), jnp.float32),
        ],
    )
    return f(x_flat, idx_t, w_t)


# speedup vs baseline: 25.8903x; 1.0882x over previous
import jax
import jax.numpy as jnp
from jax import lax
from jax.experimental import pallas as pl
from jax.experimental.pallas import tpu as pltpu
from jax.experimental.pallas import tpu_sc as plsc

_NV = 10000
_NC = 42000
_K = 3
_B = 128

_NSUB = 16
_CPT = 2640
_NCP = _NSUB * _CPT
_STEPS = _CPT // 16
_PAD = _NCP - _NC
_BUFW = 2 * _K * _CPT


def _sat_body(x_hbm, buf_hbm, out_hbm,
              x_v, buf_v, acc_v, shared, red_v, out_v, sem1, sem2):
    s = lax.axis_index("s")

    cp1 = pltpu.async_copy(x_hbm, x_v, sem1)
    cp2 = pltpu.async_copy(buf_hbm.at[pl.ds(s * _BUFW, _BUFW)], buf_v, sem2)
    cp1.wait()
    cp2.wait()

    def body(t, acc):
        off = t * 16
        pre = jnp.full((16,), float(_K - 1), dtype=jnp.float32)
        for j in range(_K):
            ij = buf_v[pl.ds(j * _CPT + off, 16)]
            vj = plsc.load_gather(x_v, [ij])
            wj = plsc.bitcast(buf_v[pl.ds((_K + j) * _CPT + off, 16)],
                              jnp.float32)
            pre = pre + wj * jnp.sign(vj)
        return acc + jnp.sign(pre)

    acc = lax.fori_loop(0, _STEPS, body, jnp.zeros((16,), jnp.float32))

    acc_v[...] = acc
    pltpu.sync_copy(acc_v, shared.at[s])
    plsc.subcore_barrier()

    @pl.when(s == 0)
    def _():
        pltpu.sync_copy(shared, red_v)
        tot = jnp.zeros((16,), jnp.float32)
        for si in range(_NSUB):
            tot = tot + red_v[si]
        total = jnp.sum(tot)
        thresh = float(_PAD + _NC - 1)
        outvec = jnp.sign(jnp.broadcast_to(total - thresh, (16,)))
        for k in range(_B // 16):
            out_v[pl.ds(k * 16, 16)] = outvec
        pltpu.sync_copy(out_v, out_hbm)


def kernel(input, emb_weight, or_weight, clause_idx):
    del input
    x_flat = emb_weight.reshape(-1)
    idx_r = (jnp.pad(clause_idx, ((0, _NCP - _NC), (0, 0)))
             .reshape(_NSUB, _CPT, _K).transpose(0, 2, 1))
    w_r = (jnp.pad(or_weight, ((0, _NCP - _NC), (0, 0)))
           .reshape(_NSUB, _CPT, _K).transpose(0, 2, 1))
    buf = jnp.concatenate(
        [idx_r, lax.bitcast_convert_type(w_r, jnp.int32)], axis=1
    ).reshape(-1)

    mesh = plsc.VectorSubcoreMesh(
        core_axis_name="c", subcore_axis_name="s", num_cores=1)
    f = pl.kernel(
        _sat_body,
        mesh=mesh,
        out_type=jax.ShapeDtypeStruct((_B,), jnp.float32),
        compiler_params=pltpu.CompilerParams(needs_layout_passes=False),
        scratch_types=[
            pltpu.VMEM((_NV,), jnp.float32),
            pltpu.VMEM((_BUFW,), jnp.int32),
            pltpu.VMEM((16,), jnp.float32),
            pltpu.VMEM_SHARED((_NSUB, 16), jnp.float32),
            pltpu.VMEM((_NSUB, 16), jnp.float32),
            pltpu.VMEM((_B,), jnp.float32),
            pltpu.SemaphoreType.DMA,
            pltpu.SemaphoreType.DMA,
        ],
    )
    return f(x_flat, buf)
